# baseline (speedup 1.0000x reference)
import functools

import jax
import jax.numpy as jnp
from jax import lax
from jax.experimental import pallas as pl
from jax.experimental.pallas import tpu as pltpu
from jax.experimental.pallas import tpu_sc as plsc

N = 10000
E = 320000
D = 128
N_CLUST = 1000

NC = 2
NS = 16
NW = NC * NS

CH = 128
GS = 16
NG = 5
EC = NG * GS
EC0 = 160
NG0 = EC0 // GS
E_PAD = NS * CH * EC0
RPT = 632
N_PAD = NS * RPT
PC = -(-N // (NW * CH))
POOL_PAD = NW * CH * PC
GPT = 64
G_PAD = NS * GPT
CW = 16

f32 = jnp.float32
i32 = jnp.int32


def _sc_mesh():
    return plsc.VectorSubcoreMesh(core_axis_name="c", subcore_axis_name="s")


def _zero_shared(sid, zero_v, dst_sh, rpt):
    nfull = rpt // CH
    for r in range(nfull):
        pltpu.sync_copy(zero_v, dst_sh.at[pl.ds(sid * rpt + r * CH, CH)])
    tail = rpt - nfull * CH
    if tail:
        pltpu.sync_copy(zero_v.at[pl.ds(0, tail)],
                        dst_sh.at[pl.ds(sid * rpt + nfull * CH, tail)])


def _copy_out(cid, sid, sh, out, buf_v, rpt):
    nfull = rpt // CH
    for r in range(nfull):
        pltpu.sync_copy(sh.at[pl.ds(sid * rpt + r * CH, CH)], buf_v)
        pltpu.sync_copy(buf_v, out.at[cid, pl.ds(sid * rpt + r * CH, CH)])
    tail = rpt - nfull * CH
    if tail:
        pltpu.sync_copy(sh.at[pl.ds(sid * rpt + nfull * CH, tail)],
                        buf_v.at[pl.ds(0, tail)])
        pltpu.sync_copy(buf_v.at[pl.ds(0, tail)],
                        out.at[cid, pl.ds(sid * rpt + nfull * CH, tail)])


EPW = EC * CH
EPG = GS * CH
PPW = PC * CH


@functools.partial(
    pl.kernel,
    out_type=(
        jax.ShapeDtypeStruct((NW, N_PAD), f32),
        jax.ShapeDtypeStruct((NW, G_PAD), f32),
    ),
    mesh=_sc_mesh(),
    scratch_types=(
        pltpu.VMEM((N_PAD,), f32),
        pltpu.VMEM((G_PAD,), f32),
        pltpu.VMEM((EPG,), i32),
        pltpu.VMEM((PPW,), i32),
    ),
    compiler_params=pltpu.CompilerParams(needs_layout_passes=False),
)
def _sc_counts(dst_hbm, clu_hbm,
               deg_out, gcnt_out,
               deg_v, gcnt_v, dst_v, clu_v):
    cid = lax.axis_index("c")
    sid = lax.axis_index("s")
    wid = cid * NS + sid
    zeros16 = jnp.zeros((16,), f32)
    ones16 = jnp.ones((16,), f32)
    pltpu.sync_copy(clu_hbm.at[wid], clu_v)

    def zb(i, c):
        deg_v[pl.ds(pl.multiple_of(i * 16, 16), 16)] = zeros16
        return c

    lax.fori_loop(0, N_PAD // 16, zb, 0)

    def zg(i, c):
        gcnt_v[pl.ds(pl.multiple_of(i * 16, 16), 16)] = zeros16
        return c

    lax.fori_loop(0, G_PAD // 16, zg, 0)

    def body(g, carry):
        pltpu.sync_copy(dst_hbm.at[wid, pl.ds(g * EPG, EPG)], dst_v)

        def sub(k, c2):
            idx = dst_v[pl.ds(pl.multiple_of(k * 16, 16), 16)]
            plsc.addupdate_scatter(deg_v, [idx], ones16)
            return c2

        return lax.fori_loop(0, EPG // 16, sub, carry)

    lax.fori_loop(0, NG, body, 0)

    def pb(k, carry):
        idx = clu_v[pl.ds(pl.multiple_of(k * 16, 16), 16)]
        plsc.addupdate_scatter(gcnt_v, [idx], ones16)
        return carry

    lax.fori_loop(0, PPW // 16, pb, 0)
    pltpu.sync_copy(deg_v, deg_out.at[wid])
    pltpu.sync_copy(gcnt_v, gcnt_out.at[wid])


def _tc_reduce_counts(c_ref, o_ref):
    o_ref[...] = jnp.sum(c_ref[...], axis=0).reshape(-1, 1)


def _agg_pipeline(tab_hbm, src_hbm, dst_hbm, sid, n_chunks, n_groups,
                  agg_sh, bufs, gsems, ssems, srcs, dsts, isems):
    pltpu.sync_copy(src_hbm.at[sid, pl.ds(0, GS)], srcs[0])
    pltpu.sync_copy(dst_hbm.at[sid, pl.ds(0, GS)], dsts[0])
    idx_desc = [None, None]
    g_desc = [None, None]
    s_desc = [None, None]
    g_desc[0] = pltpu.async_copy(tab_hbm.at[srcs[0].at[0]], bufs[0], gsems[0])
    for c in range(n_chunks):
        gi, j, b = c // GS, c % GS, c % 2
        if j == 0 and gi + 1 < n_groups:
            p = (gi + 1) % 2
            idx_desc[p] = (
                pltpu.async_copy(src_hbm.at[sid, pl.ds((gi + 1) * GS, GS)],
                                 srcs[p], isems[p][0]),
                pltpu.async_copy(dst_hbm.at[sid, pl.ds((gi + 1) * GS, GS)],
                                 dsts[p], isems[p][1]),
            )
        if c + 1 < n_chunks:
            ngi, nj, nb = (c + 1) // GS, (c + 1) % GS, (c + 1) % 2
            if nj == 0:
                for dsc in idx_desc[ngi % 2]:
                    dsc.wait()
            if s_desc[nb] is not None:
                s_desc[nb].wait()
            g_desc[nb] = pltpu.async_copy(
                tab_hbm.at[srcs[ngi % 2].at[nj]], bufs[nb], gsems[nb])
        g_desc[b].wait()
        s_desc[b] = pltpu.async_copy(
            bufs[b], agg_sh.at[dsts[gi % 2].at[j]], ssems[b], add=True)
    s_desc[(n_chunks - 1) % 2].wait()
    s_desc[n_chunks % 2].wait()


@functools.partial(
    pl.kernel,
    out_type=jax.ShapeDtypeStruct((NC, N_PAD, D), f32),
    mesh=_sc_mesh(),
    scratch_types=(
        pltpu.VMEM_SHARED((N_PAD, D), f32),
        pltpu.VMEM((GS, CH), i32),
        pltpu.VMEM((GS, CH), i32),
        pltpu.VMEM((GS, CH), i32),
        pltpu.VMEM((GS, CH), i32),
        pltpu.VMEM((CH, D), f32),
        pltpu.VMEM((CH, D), f32),
        pltpu.SemaphoreType.DMA,
        pltpu.SemaphoreType.DMA,
        pltpu.SemaphoreType.DMA,
        pltpu.SemaphoreType.DMA,
        pltpu.SemaphoreType.DMA,
        pltpu.SemaphoreType.DMA,
        pltpu.SemaphoreType.DMA,
        pltpu.SemaphoreType.DMA,
    ),
)
def _sc_agg(tab_hbm, src0_hbm, dst0_hbm, zrow_hbm,
            agg_out,
            agg_sh, src0_v, src1_v, dst0_v, dst1_v, rows0_v, rows1_v,
            sg0, sg1, ss0, ss1, sia0, sib0, sia1, sib1):
    cid = lax.axis_index("c")
    sid = lax.axis_index("s")
    pltpu.sync_copy(zrow_hbm, rows0_v)
    _zero_shared(sid, rows0_v, agg_sh, RPT)
    plsc.subcore_barrier()
    bufs = (rows0_v, rows1_v)
    gsems = (sg0, sg1)
    ssems = (ss0, ss1)
    srcs = (src0_v, src1_v)
    dsts = (dst0_v, dst1_v)
    isems = ((sia0, sib0), (sia1, sib1))

    @pl.when(cid == 0)
    def _():
        _agg_pipeline(tab_hbm, src0_hbm, dst0_hbm, sid, EC0, NG0,
                      agg_sh, bufs, gsems, ssems, srcs, dsts, isems)

    plsc.subcore_barrier()
    _copy_out(cid, sid, agg_sh, agg_out, rows0_v, RPT)


@functools.partial(
    pl.kernel,
    out_type=jax.ShapeDtypeStruct((NC, G_PAD, D), f32),
    mesh=_sc_mesh(),
    scratch_types=(
        pltpu.VMEM_SHARED((G_PAD, D), f32),
        pltpu.VMEM((PC, CH), i32),
        pltpu.VMEM((CH, D), f32),
        pltpu.VMEM((CH, D), f32),
        pltpu.SemaphoreType.DMA,
        pltpu.SemaphoreType.DMA,
        pltpu.SemaphoreType.DMA,
        pltpu.SemaphoreType.DMA,
    ),
)
def _sc_pool(h_hbm, clu_hbm, zrow_hbm,
             g_out,
             g_sh, clu_v, rows0_v, rows1_v, sg0, sg1, ss0, ss1):
    cid = lax.axis_index("c")
    sid = lax.axis_index("s")
    wid = cid * NS + sid
    pltpu.sync_copy(clu_hbm.at[wid], clu_v)
    pltpu.sync_copy(zrow_hbm, rows0_v)
    _zero_shared(sid, rows0_v, g_sh, GPT)
    plsc.subcore_barrier()
    bufs = (rows0_v, rows1_v)
    gsems = (sg0, sg1)
    ssems = (ss0, ss1)
    base = wid * PC * CH
    g_desc = [None, None]
    s_desc = [None, None]
    g_desc[0] = pltpu.async_copy(h_hbm.at[pl.ds(base, CH)], bufs[0], gsems[0])
    for c in range(PC):
        b = c % 2
        if c + 1 < PC:
            nb = (c + 1) % 2
            if s_desc[nb] is not None:
                s_desc[nb].wait()
            g_desc[nb] = pltpu.async_copy(
                h_hbm.at[pl.ds(base + (c + 1) * CH, CH)], bufs[nb], gsems[nb])
        g_desc[b].wait()
        s_desc[b] = pltpu.async_copy(
            bufs[b], g_sh.at[clu_v.at[c]], ssems[b], add=True)
    for d in s_desc:
        if d is not None:
            d.wait()
    plsc.subcore_barrier()
    _copy_out(cid, sid, g_sh, g_out, rows0_v, GPT)


def _tc_conv0(a_ref, d_ref, x_ref, wl_ref, bl_ref, wr_ref, out_ref, h_ref):
    a = a_ref[0] + a_ref[1]
    mean = a / jnp.maximum(d_ref[...], 1.0)
    out = (jnp.dot(mean, wl_ref[...], preferred_element_type=f32)
           + bl_ref[...]
           + jnp.dot(x_ref[...], wr_ref[...], preferred_element_type=f32))
    out_ref[...] = out
    h_ref[...] = jnp.maximum(out, 0.0)


def _tc_conv1(a_ref, d_ref, h_ref, wl_ref, bl_ref, wr_ref, gs_ref, gc_ref,
              y_ref, g_ref):
    a = a_ref[0] + a_ref[1]
    mean = a / jnp.maximum(d_ref[...], 1.0)
    x2 = (jnp.dot(mean, wl_ref[...], preferred_element_type=f32)
          + bl_ref[...]
          + jnp.dot(h_ref[...], wr_ref[...], preferred_element_type=f32))
    m = jnp.max(x2, axis=1, keepdims=True)
    e = x2 - m
    lse = jnp.log(jnp.sum(jnp.exp(e), axis=1, keepdims=True))
    y_ref[...] = e - lse
    gs = gs_ref[0] + gs_ref[1]
    g_ref[...] = gs / jnp.maximum(gc_ref[...], 1.0)


ROWS_BLK = 1000
GRID = N // ROWS_BLK


def kernel(x, edge_index, cluster, Wl0, bl0, Wr0, Wl1, bl1, Wr1):
    src = edge_index[0]
    dst = edge_index[1]
    e_extra = E_PAD - E
    src_f = jnp.concatenate([src, jnp.zeros((e_extra,), i32)])
    dst_pad = N + (jnp.arange(e_extra, dtype=i32) % (N_PAD - N))
    dst_f = jnp.concatenate([dst, dst_pad])
    src0 = src_f.reshape(NS, EC0, CH)
    dst0 = dst_f.reshape(NS, EC0, CH)
    dst_p = dst_f.reshape(NW, EC, CH)
    p_extra = POOL_PAD - N
    clu_pad = N_CLUST + (jnp.arange(p_extra, dtype=i32) % (G_PAD - N_CLUST))
    clu_p = jnp.concatenate([cluster.astype(i32), clu_pad]).reshape(NW, PC, CH)
    zrow = jnp.zeros((CH, D), f32)

    deg_t, gcnt_t = _sc_counts(dst_p.reshape(NW, EPW), clu_p.reshape(NW, PPW))
    cat = jnp.concatenate([deg_t, gcnt_t], axis=1)
    ncat = N_PAD + G_PAD
    red = pl.pallas_call(
        _tc_reduce_counts,
        grid=(ncat // CH,),
        in_specs=[pl.BlockSpec((NW, CH), lambda i: (0, i))],
        out_specs=pl.BlockSpec((CH, 1), lambda i: (i, 0)),
        out_shape=jax.ShapeDtypeStruct((ncat, 1), f32),
    )(cat)
    deg_c = red[:N]
    gcnt_c = red[N_PAD:N_PAD + N_CLUST]

    agg0_p = _sc_agg(x, src0, dst0, zrow)
    agg0_p = agg0_p[:, :N]

    wl0t = Wl0.T
    wr0t = Wr0.T
    bl0r = bl0.reshape(1, D)
    out, h = pl.pallas_call(
        _tc_conv0,
        grid=(GRID,),
        in_specs=[
            pl.BlockSpec((NC, ROWS_BLK, D), lambda i: (0, i, 0)),
            pl.BlockSpec((ROWS_BLK, 1), lambda i: (i, 0)),
            pl.BlockSpec((ROWS_BLK, D), lambda i: (i, 0)),
            pl.BlockSpec((D, D), lambda i: (0, 0)),
            pl.BlockSpec((1, D), lambda i: (0, 0)),
            pl.BlockSpec((D, D), lambda i: (0, 0)),
        ],
        out_specs=[
            pl.BlockSpec((ROWS_BLK, D), lambda i: (i, 0)),
            pl.BlockSpec((ROWS_BLK, D), lambda i: (i, 0)),
        ],
        out_shape=[
            jax.ShapeDtypeStruct((N, D), f32),
            jax.ShapeDtypeStruct((POOL_PAD, D), f32),
        ],
    )(agg0_p, deg_c, x, wl0t, bl0r, wr0t)

    agg1_p = _sc_agg(h, src0, dst0, zrow)
    agg1_p = agg1_p[:, :N]
    g_p = _sc_pool(h, clu_p, zrow)
    g_p = g_p[:, :N_CLUST]

    wl1t = Wl1.T
    wr1t = Wr1.T
    bl1r = bl1.reshape(1, D)
    y, g = pl.pallas_call(
        _tc_conv1,
        grid=(GRID,),
        in_specs=[
            pl.BlockSpec((NC, ROWS_BLK, D), lambda i: (0, i, 0)),
            pl.BlockSpec((ROWS_BLK, 1), lambda i: (i, 0)),
            pl.BlockSpec((ROWS_BLK, D), lambda i: (i, 0)),
            pl.BlockSpec((D, D), lambda i: (0, 0)),
            pl.BlockSpec((1, D), lambda i: (0, 0)),
            pl.BlockSpec((D, D), lambda i: (0, 0)),
            pl.BlockSpec((NC, N_CLUST, D), lambda i: (0, 0, 0)),
            pl.BlockSpec((N_CLUST, 1), lambda i: (0, 0)),
        ],
        out_specs=[
            pl.BlockSpec((ROWS_BLK, D), lambda i: (i, 0)),
            pl.BlockSpec((N_CLUST, D), lambda i: (0, 0)),
        ],
        out_shape=[
            jax.ShapeDtypeStruct((N, D), f32),
            jax.ShapeDtypeStruct((N_CLUST, D), f32),
        ],
    )(agg1_p, deg_c, h, wl1t, bl1r, wr1t, g_p, gcnt_c)

    return (y, out, g)


# speedup vs baseline: 1.1843x; 1.1843x over previous
import functools

import jax
import jax.numpy as jnp
from jax import lax
from jax.experimental import pallas as pl
from jax.experimental.pallas import tpu as pltpu
from jax.experimental.pallas import tpu_sc as plsc

N = 10000
E = 320000
D = 128
N_CLUST = 1000

NC = 2
NS = 16
NW = NC * NS

CH = 128
GS = 16
NG = 5
EC = NG * GS
EC0 = 128
EC1 = 32
NG0 = EC0 // GS
NG1 = EC1 // GS
E_PAD = NS * CH * (EC0 + EC1)
RPT = 632
N_PAD = NS * RPT
PC = -(-N // (NW * CH))
POOL_PAD = NW * CH * PC
GPT = 64
G_PAD = NS * GPT
CW = 16

f32 = jnp.float32
i32 = jnp.int32


def _sc_mesh():
    return plsc.VectorSubcoreMesh(core_axis_name="c", subcore_axis_name="s")


def _zero_shared(sid, zero_v, dst_sh, rpt):
    nfull = rpt // CH
    for r in range(nfull):
        pltpu.sync_copy(zero_v, dst_sh.at[pl.ds(sid * rpt + r * CH, CH)])
    tail = rpt - nfull * CH
    if tail:
        pltpu.sync_copy(zero_v.at[pl.ds(0, tail)],
                        dst_sh.at[pl.ds(sid * rpt + nfull * CH, tail)])


def _copy_out(cid, sid, sh, out, buf_v, rpt):
    nfull = rpt // CH
    for r in range(nfull):
        pltpu.sync_copy(sh.at[pl.ds(sid * rpt + r * CH, CH)], buf_v)
        pltpu.sync_copy(buf_v, out.at[cid, pl.ds(sid * rpt + r * CH, CH)])
    tail = rpt - nfull * CH
    if tail:
        pltpu.sync_copy(sh.at[pl.ds(sid * rpt + nfull * CH, tail)],
                        buf_v.at[pl.ds(0, tail)])
        pltpu.sync_copy(buf_v.at[pl.ds(0, tail)],
                        out.at[cid, pl.ds(sid * rpt + nfull * CH, tail)])


EPW = EC * CH
EPG = GS * CH
PPW = PC * CH


@functools.partial(
    pl.kernel,
    out_type=(
        jax.ShapeDtypeStruct((NW, N_PAD), f32),
        jax.ShapeDtypeStruct((NW, G_PAD), f32),
    ),
    mesh=_sc_mesh(),
    scratch_types=(
        pltpu.VMEM((N_PAD,), f32),
        pltpu.VMEM((G_PAD,), f32),
        pltpu.VMEM((EPG,), i32),
        pltpu.VMEM((PPW,), i32),
    ),
    compiler_params=pltpu.CompilerParams(needs_layout_passes=False),
)
def _sc_counts(dst_hbm, clu_hbm,
               deg_out, gcnt_out,
               deg_v, gcnt_v, dst_v, clu_v):
    cid = lax.axis_index("c")
    sid = lax.axis_index("s")
    wid = cid * NS + sid
    zeros16 = jnp.zeros((16,), f32)
    ones16 = jnp.ones((16,), f32)
    pltpu.sync_copy(clu_hbm.at[wid], clu_v)

    def zb(i, c):
        deg_v[pl.ds(pl.multiple_of(i * 16, 16), 16)] = zeros16
        return c

    lax.fori_loop(0, N_PAD // 16, zb, 0)

    def zg(i, c):
        gcnt_v[pl.ds(pl.multiple_of(i * 16, 16), 16)] = zeros16
        return c

    lax.fori_loop(0, G_PAD // 16, zg, 0)

    def body(g, carry):
        pltpu.sync_copy(dst_hbm.at[wid, pl.ds(g * EPG, EPG)], dst_v)

        def sub(k, c2):
            idx = dst_v[pl.ds(pl.multiple_of(k * 16, 16), 16)]
            plsc.addupdate_scatter(deg_v, [idx], ones16)
            return c2

        return lax.fori_loop(0, EPG // 16, sub, carry)

    lax.fori_loop(0, NG, body, 0)

    def pb(k, carry):
        idx = clu_v[pl.ds(pl.multiple_of(k * 16, 16), 16)]
        plsc.addupdate_scatter(gcnt_v, [idx], ones16)
        return carry

    lax.fori_loop(0, PPW // 16, pb, 0)
    pltpu.sync_copy(deg_v, deg_out.at[wid])
    pltpu.sync_copy(gcnt_v, gcnt_out.at[wid])


def _tc_reduce_counts(c_ref, o_ref):
    o_ref[...] = jnp.sum(c_ref[...], axis=0).reshape(-1, 1)


def _agg_pipeline(tab_hbm, src_hbm, dst_hbm, sid, n_chunks, n_groups,
                  agg_sh, bufs, gsems, ssems, srcs, dsts, isems):
    pltpu.sync_copy(src_hbm.at[sid, pl.ds(0, GS)], srcs[0])
    pltpu.sync_copy(dst_hbm.at[sid, pl.ds(0, GS)], dsts[0])
    idx_desc = [None, None]
    g_desc = [None, None]
    s_desc = [None, None]
    g_desc[0] = pltpu.async_copy(tab_hbm.at[srcs[0].at[0]], bufs[0], gsems[0])
    for c in range(n_chunks):
        gi, j, b = c // GS, c % GS, c % 2
        if j == 0 and gi + 1 < n_groups:
            p = (gi + 1) % 2
            idx_desc[p] = (
                pltpu.async_copy(src_hbm.at[sid, pl.ds((gi + 1) * GS, GS)],
                                 srcs[p], isems[p][0]),
                pltpu.async_copy(dst_hbm.at[sid, pl.ds((gi + 1) * GS, GS)],
                                 dsts[p], isems[p][1]),
            )
        if c + 1 < n_chunks:
            ngi, nj, nb = (c + 1) // GS, (c + 1) % GS, (c + 1) % 2
            if nj == 0:
                for dsc in idx_desc[ngi % 2]:
                    dsc.wait()
            if s_desc[nb] is not None:
                s_desc[nb].wait()
            g_desc[nb] = pltpu.async_copy(
                tab_hbm.at[srcs[ngi % 2].at[nj]], bufs[nb], gsems[nb])
        g_desc[b].wait()
        s_desc[b] = pltpu.async_copy(
            bufs[b], agg_sh.at[dsts[gi % 2].at[j]], ssems[b], add=True)
    s_desc[(n_chunks - 1) % 2].wait()
    s_desc[n_chunks % 2].wait()


@functools.partial(
    pl.kernel,
    out_type=jax.ShapeDtypeStruct((NC, N_PAD, D), f32),
    mesh=_sc_mesh(),
    scratch_types=(
        pltpu.VMEM_SHARED((N_PAD, D), f32),
        pltpu.VMEM((GS, CH), i32),
        pltpu.VMEM((GS, CH), i32),
        pltpu.VMEM((GS, CH), i32),
        pltpu.VMEM((GS, CH), i32),
        pltpu.VMEM((CH, D), f32),
        pltpu.VMEM((CH, D), f32),
        pltpu.SemaphoreType.DMA,
        pltpu.SemaphoreType.DMA,
        pltpu.SemaphoreType.DMA,
        pltpu.SemaphoreType.DMA,
        pltpu.SemaphoreType.DMA,
        pltpu.SemaphoreType.DMA,
        pltpu.SemaphoreType.DMA,
        pltpu.SemaphoreType.DMA,
    ),
)
def _sc_agg(tab_hbm, src0_hbm, dst0_hbm, src1_hbm, dst1_hbm, zrow_hbm,
            agg_out,
            agg_sh, src0_v, src1_v, dst0_v, dst1_v, rows0_v, rows1_v,
            sg0, sg1, ss0, ss1, sia0, sib0, sia1, sib1):
    cid = lax.axis_index("c")
    sid = lax.axis_index("s")
    pltpu.sync_copy(zrow_hbm, rows0_v)
    _zero_shared(sid, rows0_v, agg_sh, RPT)
    plsc.subcore_barrier()
    bufs = (rows0_v, rows1_v)
    gsems = (sg0, sg1)
    ssems = (ss0, ss1)
    srcs = (src0_v, src1_v)
    dsts = (dst0_v, dst1_v)
    isems = ((sia0, sib0), (sia1, sib1))

    @pl.when(cid == 0)
    def _():
        _agg_pipeline(tab_hbm, src0_hbm, dst0_hbm, sid, EC0, NG0,
                      agg_sh, bufs, gsems, ssems, srcs, dsts, isems)

    @pl.when(cid == 1)
    def _():
        _agg_pipeline(tab_hbm, src1_hbm, dst1_hbm, sid, EC1, NG1,
                      agg_sh, bufs, gsems, ssems, srcs, dsts, isems)

    plsc.subcore_barrier()
    _copy_out(cid, sid, agg_sh, agg_out, rows0_v, RPT)


@functools.partial(
    pl.kernel,
    out_type=jax.ShapeDtypeStruct((NC, G_PAD, D), f32),
    mesh=_sc_mesh(),
    scratch_types=(
        pltpu.VMEM_SHARED((G_PAD, D), f32),
        pltpu.VMEM((PC, CH), i32),
        pltpu.VMEM((CH, D), f32),
        pltpu.VMEM((CH, D), f32),
        pltpu.SemaphoreType.DMA,
        pltpu.SemaphoreType.DMA,
        pltpu.SemaphoreType.DMA,
        pltpu.SemaphoreType.DMA,
    ),
)
def _sc_pool(h_hbm, clu_hbm, zrow_hbm,
             g_out,
             g_sh, clu_v, rows0_v, rows1_v, sg0, sg1, ss0, ss1):
    cid = lax.axis_index("c")
    sid = lax.axis_index("s")
    wid = cid * NS + sid
    pltpu.sync_copy(clu_hbm.at[wid], clu_v)
    pltpu.sync_copy(zrow_hbm, rows0_v)
    _zero_shared(sid, rows0_v, g_sh, GPT)
    plsc.subcore_barrier()
    bufs = (rows0_v, rows1_v)
    gsems = (sg0, sg1)
    ssems = (ss0, ss1)
    base = wid * PC * CH
    g_desc = [None, None]
    s_desc = [None, None]
    g_desc[0] = pltpu.async_copy(h_hbm.at[pl.ds(base, CH)], bufs[0], gsems[0])
    for c in range(PC):
        b = c % 2
        if c + 1 < PC:
            nb = (c + 1) % 2
            if s_desc[nb] is not None:
                s_desc[nb].wait()
            g_desc[nb] = pltpu.async_copy(
                h_hbm.at[pl.ds(base + (c + 1) * CH, CH)], bufs[nb], gsems[nb])
        g_desc[b].wait()
        s_desc[b] = pltpu.async_copy(
            bufs[b], g_sh.at[clu_v.at[c]], ssems[b], add=True)
    for d in s_desc:
        if d is not None:
            d.wait()
    plsc.subcore_barrier()
    _copy_out(cid, sid, g_sh, g_out, rows0_v, GPT)


def _tc_conv0(a_ref, d_ref, x_ref, wl_ref, bl_ref, wr_ref, out_ref, h_ref):
    a = a_ref[0] + a_ref[1]
    mean = a / jnp.maximum(d_ref[...], 1.0)
    out = (jnp.dot(mean, wl_ref[...], preferred_element_type=f32)
           + bl_ref[...]
           + jnp.dot(x_ref[...], wr_ref[...], preferred_element_type=f32))
    out_ref[...] = out
    h_ref[...] = jnp.maximum(out, 0.0)


def _tc_conv1(a_ref, d_ref, h_ref, wl_ref, bl_ref, wr_ref, gs_ref, gc_ref,
              y_ref, g_ref):
    a = a_ref[0] + a_ref[1]
    mean = a / jnp.maximum(d_ref[...], 1.0)
    x2 = (jnp.dot(mean, wl_ref[...], preferred_element_type=f32)
          + bl_ref[...]
          + jnp.dot(h_ref[...], wr_ref[...], preferred_element_type=f32))
    m = jnp.max(x2, axis=1, keepdims=True)
    e = x2 - m
    lse = jnp.log(jnp.sum(jnp.exp(e), axis=1, keepdims=True))
    y_ref[...] = e - lse
    gs = gs_ref[0] + gs_ref[1]
    g_ref[...] = gs / jnp.maximum(gc_ref[...], 1.0)


ROWS_BLK = 1000
GRID = N // ROWS_BLK


def kernel(x, edge_index, cluster, Wl0, bl0, Wr0, Wl1, bl1, Wr1):
    src = edge_index[0]
    dst = edge_index[1]
    e_extra = E_PAD - E
    src_f = jnp.concatenate([src, jnp.zeros((e_extra,), i32)])
    dst_pad = N + (jnp.arange(e_extra, dtype=i32) % (N_PAD - N))
    dst_f = jnp.concatenate([dst, dst_pad])
    split = NS * EC0 * CH
    src0 = src_f[:split].reshape(NS, EC0, CH)
    dst0 = dst_f[:split].reshape(NS, EC0, CH)
    src1 = src_f[split:].reshape(NS, EC1, CH)
    dst1 = dst_f[split:].reshape(NS, EC1, CH)
    dst_p = dst_f.reshape(NW, EC, CH)
    p_extra = POOL_PAD - N
    clu_pad = N_CLUST + (jnp.arange(p_extra, dtype=i32) % (G_PAD - N_CLUST))
    clu_p = jnp.concatenate([cluster.astype(i32), clu_pad]).reshape(NW, PC, CH)
    zrow = jnp.zeros((CH, D), f32)

    deg_t, gcnt_t = _sc_counts(dst_p.reshape(NW, EPW), clu_p.reshape(NW, PPW))
    cat = jnp.concatenate([deg_t, gcnt_t], axis=1)
    ncat = N_PAD + G_PAD
    red = pl.pallas_call(
        _tc_reduce_counts,
        grid=(ncat // CH,),
        in_specs=[pl.BlockSpec((NW, CH), lambda i: (0, i))],
        out_specs=pl.BlockSpec((CH, 1), lambda i: (i, 0)),
        out_shape=jax.ShapeDtypeStruct((ncat, 1), f32),
    )(cat)
    deg_c = red[:N]
    gcnt_c = red[N_PAD:N_PAD + N_CLUST]

    agg0_p = _sc_agg(x, src0, dst0, src1, dst1, zrow)
    agg0_p = agg0_p[:, :N]

    wl0t = Wl0.T
    wr0t = Wr0.T
    bl0r = bl0.reshape(1, D)
    out, h = pl.pallas_call(
        _tc_conv0,
        grid=(GRID,),
        in_specs=[
            pl.BlockSpec((NC, ROWS_BLK, D), lambda i: (0, i, 0)),
            pl.BlockSpec((ROWS_BLK, 1), lambda i: (i, 0)),
            pl.BlockSpec((ROWS_BLK, D), lambda i: (i, 0)),
            pl.BlockSpec((D, D), lambda i: (0, 0)),
            pl.BlockSpec((1, D), lambda i: (0, 0)),
            pl.BlockSpec((D, D), lambda i: (0, 0)),
        ],
        out_specs=[
            pl.BlockSpec((ROWS_BLK, D), lambda i: (i, 0)),
            pl.BlockSpec((ROWS_BLK, D), lambda i: (i, 0)),
        ],
        out_shape=[
            jax.ShapeDtypeStruct((N, D), f32),
            jax.ShapeDtypeStruct((POOL_PAD, D), f32),
        ],
    )(agg0_p, deg_c, x, wl0t, bl0r, wr0t)

    agg1_p = _sc_agg(h, src0, dst0, src1, dst1, zrow)
    agg1_p = agg1_p[:, :N]
    g_p = _sc_pool(h, clu_p, zrow)
    g_p = g_p[:, :N_CLUST]

    wl1t = Wl1.T
    wr1t = Wr1.T
    bl1r = bl1.reshape(1, D)
    y, g = pl.pallas_call(
        _tc_conv1,
        grid=(GRID,),
        in_specs=[
            pl.BlockSpec((NC, ROWS_BLK, D), lambda i: (0, i, 0)),
            pl.BlockSpec((ROWS_BLK, 1), lambda i: (i, 0)),
            pl.BlockSpec((ROWS_BLK, D), lambda i: (i, 0)),
            pl.BlockSpec((D, D), lambda i: (0, 0)),
            pl.BlockSpec((1, D), lambda i: (0, 0)),
            pl.BlockSpec((D, D), lambda i: (0, 0)),
            pl.BlockSpec((NC, N_CLUST, D), lambda i: (0, 0, 0)),
            pl.BlockSpec((N_CLUST, 1), lambda i: (0, 0)),
        ],
        out_specs=[
            pl.BlockSpec((ROWS_BLK, D), lambda i: (i, 0)),
            pl.BlockSpec((N_CLUST, D), lambda i: (0, 0)),
        ],
        out_shape=[
            jax.ShapeDtypeStruct((N, D), f32),
            jax.ShapeDtypeStruct((N_CLUST, D), f32),
        ],
    )(agg1_p, deg_c, h, wl1t, bl1r, wr1t, g_p, gcnt_c)

    return (y, out, g)


# speedup vs baseline: 1.3037x; 1.1008x over previous
import functools

import jax
import jax.numpy as jnp
from jax import lax
from jax.experimental import pallas as pl
from jax.experimental.pallas import tpu as pltpu
from jax.experimental.pallas import tpu_sc as plsc

N = 10000
E = 320000
D = 128
N_CLUST = 1000

NC = 2
NS = 16
NW = NC * NS

CH = 128
GS = 16
NG = 5
EC = NG * GS
EC0 = 144
EC1 = 16
NG0 = EC0 // GS
NG1 = EC1 // GS
E_PAD = NS * CH * (EC0 + EC1)
RPT = 632
N_PAD = NS * RPT
PC = -(-N // (NW * CH))
POOL_PAD = NW * CH * PC
GPT = 64
G_PAD = NS * GPT
CW = 16

f32 = jnp.float32
i32 = jnp.int32


def _sc_mesh():
    return plsc.VectorSubcoreMesh(core_axis_name="c", subcore_axis_name="s")


def _zero_shared(sid, zero_v, dst_sh, rpt):
    nfull = rpt // CH
    for r in range(nfull):
        pltpu.sync_copy(zero_v, dst_sh.at[pl.ds(sid * rpt + r * CH, CH)])
    tail = rpt - nfull * CH
    if tail:
        pltpu.sync_copy(zero_v.at[pl.ds(0, tail)],
                        dst_sh.at[pl.ds(sid * rpt + nfull * CH, tail)])


def _copy_out(cid, sid, sh, out, buf_v, rpt):
    nfull = rpt // CH
    for r in range(nfull):
        pltpu.sync_copy(sh.at[pl.ds(sid * rpt + r * CH, CH)], buf_v)
        pltpu.sync_copy(buf_v, out.at[cid, pl.ds(sid * rpt + r * CH, CH)])
    tail = rpt - nfull * CH
    if tail:
        pltpu.sync_copy(sh.at[pl.ds(sid * rpt + nfull * CH, tail)],
                        buf_v.at[pl.ds(0, tail)])
        pltpu.sync_copy(buf_v.at[pl.ds(0, tail)],
                        out.at[cid, pl.ds(sid * rpt + nfull * CH, tail)])


EPW = EC * CH
EPG = GS * CH
PPW = PC * CH


@functools.partial(
    pl.kernel,
    out_type=(
        jax.ShapeDtypeStruct((NW, N_PAD), f32),
        jax.ShapeDtypeStruct((NW, G_PAD), f32),
    ),
    mesh=_sc_mesh(),
    scratch_types=(
        pltpu.VMEM((N_PAD,), f32),
        pltpu.VMEM((G_PAD,), f32),
        pltpu.VMEM((EPG,), i32),
        pltpu.VMEM((PPW,), i32),
    ),
    compiler_params=pltpu.CompilerParams(needs_layout_passes=False),
)
def _sc_counts(dst_hbm, clu_hbm,
               deg_out, gcnt_out,
               deg_v, gcnt_v, dst_v, clu_v):
    cid = lax.axis_index("c")
    sid = lax.axis_index("s")
    wid = cid * NS + sid
    zeros16 = jnp.zeros((16,), f32)
    ones16 = jnp.ones((16,), f32)
    pltpu.sync_copy(clu_hbm.at[wid], clu_v)

    def zb(i, c):
        deg_v[pl.ds(pl.multiple_of(i * 16, 16), 16)] = zeros16
        return c

    lax.fori_loop(0, N_PAD // 16, zb, 0)

    def zg(i, c):
        gcnt_v[pl.ds(pl.multiple_of(i * 16, 16), 16)] = zeros16
        return c

    lax.fori_loop(0, G_PAD // 16, zg, 0)

    def body(g, carry):
        pltpu.sync_copy(dst_hbm.at[wid, pl.ds(g * EPG, EPG)], dst_v)

        def sub(k, c2):
            idx = dst_v[pl.ds(pl.multiple_of(k * 16, 16), 16)]
            plsc.addupdate_scatter(deg_v, [idx], ones16)
            return c2

        return lax.fori_loop(0, EPG // 16, sub, carry)

    lax.fori_loop(0, NG, body, 0)

    def pb(k, carry):
        idx = clu_v[pl.ds(pl.multiple_of(k * 16, 16), 16)]
        plsc.addupdate_scatter(gcnt_v, [idx], ones16)
        return carry

    lax.fori_loop(0, PPW // 16, pb, 0)
    pltpu.sync_copy(deg_v, deg_out.at[wid])
    pltpu.sync_copy(gcnt_v, gcnt_out.at[wid])


def _tc_reduce_counts(c_ref, o_ref):
    o_ref[...] = jnp.sum(c_ref[...], axis=0).reshape(-1, 1)


def _agg_pipeline(tab_hbm, src_hbm, dst_hbm, sid, n_chunks, n_groups,
                  agg_sh, bufs, gsems, ssems, srcs, dsts, isems):
    pltpu.sync_copy(src_hbm.at[sid, pl.ds(0, GS)], srcs[0])
    pltpu.sync_copy(dst_hbm.at[sid, pl.ds(0, GS)], dsts[0])
    idx_desc = [None, None]
    g_desc = [None, None]
    s_desc = [None, None]
    g_desc[0] = pltpu.async_copy(tab_hbm.at[srcs[0].at[0]], bufs[0], gsems[0])
    for c in range(n_chunks):
        gi, j, b = c // GS, c % GS, c % 2
        if j == 0 and gi + 1 < n_groups:
            p = (gi + 1) % 2
            idx_desc[p] = (
                pltpu.async_copy(src_hbm.at[sid, pl.ds((gi + 1) * GS, GS)],
                                 srcs[p], isems[p][0]),
                pltpu.async_copy(dst_hbm.at[sid, pl.ds((gi + 1) * GS, GS)],
                                 dsts[p], isems[p][1]),
            )
        if c + 1 < n_chunks:
            ngi, nj, nb = (c + 1) // GS, (c + 1) % GS, (c + 1) % 2
            if nj == 0:
                for dsc in idx_desc[ngi % 2]:
                    dsc.wait()
            if s_desc[nb] is not None:
                s_desc[nb].wait()
            g_desc[nb] = pltpu.async_copy(
                tab_hbm.at[srcs[ngi % 2].at[nj]], bufs[nb], gsems[nb])
        g_desc[b].wait()
        s_desc[b] = pltpu.async_copy(
            bufs[b], agg_sh.at[dsts[gi % 2].at[j]], ssems[b], add=True)
    s_desc[(n_chunks - 1) % 2].wait()
    s_desc[n_chunks % 2].wait()


@functools.partial(
    pl.kernel,
    out_type=jax.ShapeDtypeStruct((NC, N_PAD, D), f32),
    mesh=_sc_mesh(),
    scratch_types=(
        pltpu.VMEM_SHARED((N_PAD, D), f32),
        pltpu.VMEM((GS, CH), i32),
        pltpu.VMEM((GS, CH), i32),
        pltpu.VMEM((GS, CH), i32),
        pltpu.VMEM((GS, CH), i32),
        pltpu.VMEM((CH, D), f32),
        pltpu.VMEM((CH, D), f32),
        pltpu.SemaphoreType.DMA,
        pltpu.SemaphoreType.DMA,
        pltpu.SemaphoreType.DMA,
        pltpu.SemaphoreType.DMA,
        pltpu.SemaphoreType.DMA,
        pltpu.SemaphoreType.DMA,
        pltpu.SemaphoreType.DMA,
        pltpu.SemaphoreType.DMA,
    ),
)
def _sc_agg(tab_hbm, src0_hbm, dst0_hbm, src1_hbm, dst1_hbm, zrow_hbm,
            agg_out,
            agg_sh, src0_v, src1_v, dst0_v, dst1_v, rows0_v, rows1_v,
            sg0, sg1, ss0, ss1, sia0, sib0, sia1, sib1):
    cid = lax.axis_index("c")
    sid = lax.axis_index("s")
    pltpu.sync_copy(zrow_hbm, rows0_v)
    _zero_shared(sid, rows0_v, agg_sh, RPT)
    plsc.subcore_barrier()
    bufs = (rows0_v, rows1_v)
    gsems = (sg0, sg1)
    ssems = (ss0, ss1)
    srcs = (src0_v, src1_v)
    dsts = (dst0_v, dst1_v)
    isems = ((sia0, sib0), (sia1, sib1))

    @pl.when(cid == 0)
    def _():
        _agg_pipeline(tab_hbm, src0_hbm, dst0_hbm, sid, EC0, NG0,
                      agg_sh, bufs, gsems, ssems, srcs, dsts, isems)

    @pl.when(cid == 1)
    def _():
        _agg_pipeline(tab_hbm, src1_hbm, dst1_hbm, sid, EC1, NG1,
                      agg_sh, bufs, gsems, ssems, srcs, dsts, isems)

    plsc.subcore_barrier()
    _copy_out(cid, sid, agg_sh, agg_out, rows0_v, RPT)


@functools.partial(
    pl.kernel,
    out_type=jax.ShapeDtypeStruct((NC, G_PAD, D), f32),
    mesh=_sc_mesh(),
    scratch_types=(
        pltpu.VMEM_SHARED((G_PAD, D), f32),
        pltpu.VMEM((PC, CH), i32),
        pltpu.VMEM((CH, D), f32),
        pltpu.VMEM((CH, D), f32),
        pltpu.SemaphoreType.DMA,
        pltpu.SemaphoreType.DMA,
        pltpu.SemaphoreType.DMA,
        pltpu.SemaphoreType.DMA,
    ),
)
def _sc_pool(h_hbm, clu_hbm, zrow_hbm,
             g_out,
             g_sh, clu_v, rows0_v, rows1_v, sg0, sg1, ss0, ss1):
    cid = lax.axis_index("c")
    sid = lax.axis_index("s")
    wid = cid * NS + sid
    pltpu.sync_copy(clu_hbm.at[wid], clu_v)
    pltpu.sync_copy(zrow_hbm, rows0_v)
    _zero_shared(sid, rows0_v, g_sh, GPT)
    plsc.subcore_barrier()
    bufs = (rows0_v, rows1_v)
    gsems = (sg0, sg1)
    ssems = (ss0, ss1)
    base = wid * PC * CH
    g_desc = [None, None]
    s_desc = [None, None]
    g_desc[0] = pltpu.async_copy(h_hbm.at[pl.ds(base, CH)], bufs[0], gsems[0])
    for c in range(PC):
        b = c % 2
        if c + 1 < PC:
            nb = (c + 1) % 2
            if s_desc[nb] is not None:
                s_desc[nb].wait()
            g_desc[nb] = pltpu.async_copy(
                h_hbm.at[pl.ds(base + (c + 1) * CH, CH)], bufs[nb], gsems[nb])
        g_desc[b].wait()
        s_desc[b] = pltpu.async_copy(
            bufs[b], g_sh.at[clu_v.at[c]], ssems[b], add=True)
    for d in s_desc:
        if d is not None:
            d.wait()
    plsc.subcore_barrier()
    _copy_out(cid, sid, g_sh, g_out, rows0_v, GPT)


def _tc_conv0(a_ref, d_ref, x_ref, wl_ref, bl_ref, wr_ref, out_ref, h_ref):
    a = a_ref[0] + a_ref[1]
    mean = a / jnp.maximum(d_ref[...], 1.0)
    out = (jnp.dot(mean, wl_ref[...], preferred_element_type=f32)
           + bl_ref[...]
           + jnp.dot(x_ref[...], wr_ref[...], preferred_element_type=f32))
    out_ref[...] = out
    h_ref[...] = jnp.maximum(out, 0.0)


def _tc_conv1(a_ref, d_ref, h_ref, wl_ref, bl_ref, wr_ref, gs_ref, gc_ref,
              y_ref, g_ref):
    a = a_ref[0] + a_ref[1]
    mean = a / jnp.maximum(d_ref[...], 1.0)
    x2 = (jnp.dot(mean, wl_ref[...], preferred_element_type=f32)
          + bl_ref[...]
          + jnp.dot(h_ref[...], wr_ref[...], preferred_element_type=f32))
    m = jnp.max(x2, axis=1, keepdims=True)
    e = x2 - m
    lse = jnp.log(jnp.sum(jnp.exp(e), axis=1, keepdims=True))
    y_ref[...] = e - lse
    gs = gs_ref[0] + gs_ref[1]
    g_ref[...] = gs / jnp.maximum(gc_ref[...], 1.0)


ROWS_BLK = 1000
GRID = N // ROWS_BLK


def kernel(x, edge_index, cluster, Wl0, bl0, Wr0, Wl1, bl1, Wr1):
    src = edge_index[0]
    dst = edge_index[1]
    e_extra = E_PAD - E
    src_f = jnp.concatenate([src, jnp.zeros((e_extra,), i32)])
    dst_pad = N + (jnp.arange(e_extra, dtype=i32) % (N_PAD - N))
    dst_f = jnp.concatenate([dst, dst_pad])
    split = NS * EC0 * CH
    src0 = src_f[:split].reshape(NS, EC0, CH)
    dst0 = dst_f[:split].reshape(NS, EC0, CH)
    src1 = src_f[split:].reshape(NS, EC1, CH)
    dst1 = dst_f[split:].reshape(NS, EC1, CH)
    dst_p = dst_f.reshape(NW, EC, CH)
    p_extra = POOL_PAD - N
    clu_pad = N_CLUST + (jnp.arange(p_extra, dtype=i32) % (G_PAD - N_CLUST))
    clu_p = jnp.concatenate([cluster.astype(i32), clu_pad]).reshape(NW, PC, CH)
    zrow = jnp.zeros((CH, D), f32)

    deg_t, gcnt_t = _sc_counts(dst_p.reshape(NW, EPW), clu_p.reshape(NW, PPW))
    cat = jnp.concatenate([deg_t, gcnt_t], axis=1)
    ncat = N_PAD + G_PAD
    red = pl.pallas_call(
        _tc_reduce_counts,
        grid=(ncat // CH,),
        in_specs=[pl.BlockSpec((NW, CH), lambda i: (0, i))],
        out_specs=pl.BlockSpec((CH, 1), lambda i: (i, 0)),
        out_shape=jax.ShapeDtypeStruct((ncat, 1), f32),
    )(cat)
    deg_c = red[:N]
    gcnt_c = red[N_PAD:N_PAD + N_CLUST]

    agg0_p = _sc_agg(x, src0, dst0, src1, dst1, zrow)
    agg0_p = agg0_p[:, :N]

    wl0t = Wl0.T
    wr0t = Wr0.T
    bl0r = bl0.reshape(1, D)
    out, h = pl.pallas_call(
        _tc_conv0,
        grid=(GRID,),
        in_specs=[
            pl.BlockSpec((NC, ROWS_BLK, D), lambda i: (0, i, 0)),
            pl.BlockSpec((ROWS_BLK, 1), lambda i: (i, 0)),
            pl.BlockSpec((ROWS_BLK, D), lambda i: (i, 0)),
            pl.BlockSpec((D, D), lambda i: (0, 0)),
            pl.BlockSpec((1, D), lambda i: (0, 0)),
            pl.BlockSpec((D, D), lambda i: (0, 0)),
        ],
        out_specs=[
            pl.BlockSpec((ROWS_BLK, D), lambda i: (i, 0)),
            pl.BlockSpec((ROWS_BLK, D), lambda i: (i, 0)),
        ],
        out_shape=[
            jax.ShapeDtypeStruct((N, D), f32),
            jax.ShapeDtypeStruct((POOL_PAD, D), f32),
        ],
    )(agg0_p, deg_c, x, wl0t, bl0r, wr0t)

    agg1_p = _sc_agg(h, src0, dst0, src1, dst1, zrow)
    agg1_p = agg1_p[:, :N]
    g_p = _sc_pool(h, clu_p, zrow)
    g_p = g_p[:, :N_CLUST]

    wl1t = Wl1.T
    wr1t = Wr1.T
    bl1r = bl1.reshape(1, D)
    y, g = pl.pallas_call(
        _tc_conv1,
        grid=(GRID,),
        in_specs=[
            pl.BlockSpec((NC, ROWS_BLK, D), lambda i: (0, i, 0)),
            pl.BlockSpec((ROWS_BLK, 1), lambda i: (i, 0)),
            pl.BlockSpec((ROWS_BLK, D), lambda i: (i, 0)),
            pl.BlockSpec((D, D), lambda i: (0, 0)),
            pl.BlockSpec((1, D), lambda i: (0, 0)),
            pl.BlockSpec((D, D), lambda i: (0, 0)),
            pl.BlockSpec((NC, N_CLUST, D), lambda i: (0, 0, 0)),
            pl.BlockSpec((N_CLUST, 1), lambda i: (0, 0)),
        ],
        out_specs=[
            pl.BlockSpec((ROWS_BLK, D), lambda i: (i, 0)),
            pl.BlockSpec((N_CLUST, D), lambda i: (0, 0)),
        ],
        out_shape=[
            jax.ShapeDtypeStruct((N, D), f32),
            jax.ShapeDtypeStruct((N_CLUST, D), f32),
        ],
    )(agg1_p, deg_c, h, wl1t, bl1r, wr1t, g_p, gcnt_c)

    return (y, out, g)


# speedup vs baseline: 1.3298x; 1.0200x over previous
import functools

import jax
import jax.numpy as jnp
from jax import lax
from jax.experimental import pallas as pl
from jax.experimental.pallas import tpu as pltpu
from jax.experimental.pallas import tpu_sc as plsc

N = 10000
E = 320000
D = 128
N_CLUST = 1000

NC = 2
NS = 16
NW = NC * NS

CH = 128
GS = 16
NG = 5
EC = NG * GS
EC0 = 152
EC1 = 8
NG0 = EC0 // GS
NG1 = EC1 // GS
E_PAD = NS * CH * (EC0 + EC1)
RPT = 632
N_PAD = NS * RPT
PC = -(-N // (NW * CH))
POOL_PAD = NW * CH * PC
GPT = 64
G_PAD = NS * GPT
CW = 16

f32 = jnp.float32
i32 = jnp.int32


def _sc_mesh():
    return plsc.VectorSubcoreMesh(core_axis_name="c", subcore_axis_name="s")


def _zero_shared(sid, zero_v, dst_sh, rpt):
    nfull = rpt // CH
    for r in range(nfull):
        pltpu.sync_copy(zero_v, dst_sh.at[pl.ds(sid * rpt + r * CH, CH)])
    tail = rpt - nfull * CH
    if tail:
        pltpu.sync_copy(zero_v.at[pl.ds(0, tail)],
                        dst_sh.at[pl.ds(sid * rpt + nfull * CH, tail)])


def _copy_out(cid, sid, sh, out, buf_v, rpt):
    nfull = rpt // CH
    for r in range(nfull):
        pltpu.sync_copy(sh.at[pl.ds(sid * rpt + r * CH, CH)], buf_v)
        pltpu.sync_copy(buf_v, out.at[cid, pl.ds(sid * rpt + r * CH, CH)])
    tail = rpt - nfull * CH
    if tail:
        pltpu.sync_copy(sh.at[pl.ds(sid * rpt + nfull * CH, tail)],
                        buf_v.at[pl.ds(0, tail)])
        pltpu.sync_copy(buf_v.at[pl.ds(0, tail)],
                        out.at[cid, pl.ds(sid * rpt + nfull * CH, tail)])


EPW = EC * CH
EPG = GS * CH
PPW = PC * CH


@functools.partial(
    pl.kernel,
    out_type=(
        jax.ShapeDtypeStruct((NW, N_PAD), f32),
        jax.ShapeDtypeStruct((NW, G_PAD), f32),
    ),
    mesh=_sc_mesh(),
    scratch_types=(
        pltpu.VMEM((N_PAD,), f32),
        pltpu.VMEM((G_PAD,), f32),
        pltpu.VMEM((EPG,), i32),
        pltpu.VMEM((PPW,), i32),
    ),
    compiler_params=pltpu.CompilerParams(needs_layout_passes=False),
)
def _sc_counts(dst_hbm, clu_hbm,
               deg_out, gcnt_out,
               deg_v, gcnt_v, dst_v, clu_v):
    cid = lax.axis_index("c")
    sid = lax.axis_index("s")
    wid = cid * NS + sid
    zeros16 = jnp.zeros((16,), f32)
    ones16 = jnp.ones((16,), f32)
    pltpu.sync_copy(clu_hbm.at[wid], clu_v)

    def zb(i, c):
        deg_v[pl.ds(pl.multiple_of(i * 16, 16), 16)] = zeros16
        return c

    lax.fori_loop(0, N_PAD // 16, zb, 0)

    def zg(i, c):
        gcnt_v[pl.ds(pl.multiple_of(i * 16, 16), 16)] = zeros16
        return c

    lax.fori_loop(0, G_PAD // 16, zg, 0)

    def body(g, carry):
        pltpu.sync_copy(dst_hbm.at[wid, pl.ds(g * EPG, EPG)], dst_v)

        def sub(k, c2):
            idx = dst_v[pl.ds(pl.multiple_of(k * 16, 16), 16)]
            plsc.addupdate_scatter(deg_v, [idx], ones16)
            return c2

        return lax.fori_loop(0, EPG // 16, sub, carry)

    lax.fori_loop(0, NG, body, 0)

    def pb(k, carry):
        idx = clu_v[pl.ds(pl.multiple_of(k * 16, 16), 16)]
        plsc.addupdate_scatter(gcnt_v, [idx], ones16)
        return carry

    lax.fori_loop(0, PPW // 16, pb, 0)
    pltpu.sync_copy(deg_v, deg_out.at[wid])
    pltpu.sync_copy(gcnt_v, gcnt_out.at[wid])


def _tc_reduce_counts(c_ref, o_ref):
    o_ref[...] = jnp.sum(c_ref[...], axis=0).reshape(-1, 1)


def _agg_pipeline(tab_hbm, src_hbm, dst_hbm, sid, n_chunks,
                  agg_sh, bufs, gsems, ssems, srcs, dsts, isems):
    n_groups = -(-n_chunks // GS)
    g0 = min(GS, n_chunks)
    pltpu.sync_copy(src_hbm.at[sid, pl.ds(0, g0)], srcs[0].at[pl.ds(0, g0)])
    pltpu.sync_copy(dst_hbm.at[sid, pl.ds(0, g0)], dsts[0].at[pl.ds(0, g0)])
    idx_desc = [None, None]
    g_desc = [None, None]
    s_desc = [None, None]
    g_desc[0] = pltpu.async_copy(tab_hbm.at[srcs[0].at[0]], bufs[0], gsems[0])
    for c in range(n_chunks):
        gi, j, b = c // GS, c % GS, c % 2
        if j == 0 and gi + 1 < n_groups:
            p = (gi + 1) % 2
            psz = min(GS, n_chunks - (gi + 1) * GS)
            idx_desc[p] = (
                pltpu.async_copy(src_hbm.at[sid, pl.ds((gi + 1) * GS, psz)],
                                 srcs[p].at[pl.ds(0, psz)], isems[p][0]),
                pltpu.async_copy(dst_hbm.at[sid, pl.ds((gi + 1) * GS, psz)],
                                 dsts[p].at[pl.ds(0, psz)], isems[p][1]),
            )
        if c + 1 < n_chunks:
            ngi, nj, nb = (c + 1) // GS, (c + 1) % GS, (c + 1) % 2
            if nj == 0:
                for dsc in idx_desc[ngi % 2]:
                    dsc.wait()
            if s_desc[nb] is not None:
                s_desc[nb].wait()
            g_desc[nb] = pltpu.async_copy(
                tab_hbm.at[srcs[ngi % 2].at[nj]], bufs[nb], gsems[nb])
        g_desc[b].wait()
        s_desc[b] = pltpu.async_copy(
            bufs[b], agg_sh.at[dsts[gi % 2].at[j]], ssems[b], add=True)
    s_desc[(n_chunks - 1) % 2].wait()
    s_desc[n_chunks % 2].wait()


@functools.partial(
    pl.kernel,
    out_type=jax.ShapeDtypeStruct((NC, N_PAD, D), f32),
    mesh=_sc_mesh(),
    scratch_types=(
        pltpu.VMEM_SHARED((N_PAD, D), f32),
        pltpu.VMEM((GS, CH), i32),
        pltpu.VMEM((GS, CH), i32),
        pltpu.VMEM((GS, CH), i32),
        pltpu.VMEM((GS, CH), i32),
        pltpu.VMEM((CH, D), f32),
        pltpu.VMEM((CH, D), f32),
        pltpu.SemaphoreType.DMA,
        pltpu.SemaphoreType.DMA,
        pltpu.SemaphoreType.DMA,
        pltpu.SemaphoreType.DMA,
        pltpu.SemaphoreType.DMA,
        pltpu.SemaphoreType.DMA,
        pltpu.SemaphoreType.DMA,
        pltpu.SemaphoreType.DMA,
    ),
)
def _sc_agg(tab_hbm, src0_hbm, dst0_hbm, src1_hbm, dst1_hbm, zrow_hbm,
            agg_out,
            agg_sh, src0_v, src1_v, dst0_v, dst1_v, rows0_v, rows1_v,
            sg0, sg1, ss0, ss1, sia0, sib0, sia1, sib1):
    cid = lax.axis_index("c")
    sid = lax.axis_index("s")
    pltpu.sync_copy(zrow_hbm, rows0_v)
    _zero_shared(sid, rows0_v, agg_sh, RPT)
    plsc.subcore_barrier()
    bufs = (rows0_v, rows1_v)
    gsems = (sg0, sg1)
    ssems = (ss0, ss1)
    srcs = (src0_v, src1_v)
    dsts = (dst0_v, dst1_v)
    isems = ((sia0, sib0), (sia1, sib1))

    @pl.when(cid == 0)
    def _():
        _agg_pipeline(tab_hbm, src0_hbm, dst0_hbm, sid, EC0,
                      agg_sh, bufs, gsems, ssems, srcs, dsts, isems)

    @pl.when(cid == 1)
    def _():
        _agg_pipeline(tab_hbm, src1_hbm, dst1_hbm, sid, EC1,
                      agg_sh, bufs, gsems, ssems, srcs, dsts, isems)

    plsc.subcore_barrier()
    _copy_out(cid, sid, agg_sh, agg_out, rows0_v, RPT)


@functools.partial(
    pl.kernel,
    out_type=jax.ShapeDtypeStruct((NC, G_PAD, D), f32),
    mesh=_sc_mesh(),
    scratch_types=(
        pltpu.VMEM_SHARED((G_PAD, D), f32),
        pltpu.VMEM((PC, CH), i32),
        pltpu.VMEM((CH, D), f32),
        pltpu.VMEM((CH, D), f32),
        pltpu.SemaphoreType.DMA,
        pltpu.SemaphoreType.DMA,
        pltpu.SemaphoreType.DMA,
        pltpu.SemaphoreType.DMA,
    ),
)
def _sc_pool(h_hbm, clu_hbm, zrow_hbm,
             g_out,
             g_sh, clu_v, rows0_v, rows1_v, sg0, sg1, ss0, ss1):
    cid = lax.axis_index("c")
    sid = lax.axis_index("s")
    wid = cid * NS + sid
    pltpu.sync_copy(clu_hbm.at[wid], clu_v)
    pltpu.sync_copy(zrow_hbm, rows0_v)
    _zero_shared(sid, rows0_v, g_sh, GPT)
    plsc.subcore_barrier()
    bufs = (rows0_v, rows1_v)
    gsems = (sg0, sg1)
    ssems = (ss0, ss1)
    base = wid * PC * CH
    g_desc = [None, None]
    s_desc = [None, None]
    g_desc[0] = pltpu.async_copy(h_hbm.at[pl.ds(base, CH)], bufs[0], gsems[0])
    for c in range(PC):
        b = c % 2
        if c + 1 < PC:
            nb = (c + 1) % 2
            if s_desc[nb] is not None:
                s_desc[nb].wait()
            g_desc[nb] = pltpu.async_copy(
                h_hbm.at[pl.ds(base + (c + 1) * CH, CH)], bufs[nb], gsems[nb])
        g_desc[b].wait()
        s_desc[b] = pltpu.async_copy(
            bufs[b], g_sh.at[clu_v.at[c]], ssems[b], add=True)
    for d in s_desc:
        if d is not None:
            d.wait()
    plsc.subcore_barrier()
    _copy_out(cid, sid, g_sh, g_out, rows0_v, GPT)


def _tc_conv0(a_ref, d_ref, x_ref, wl_ref, bl_ref, wr_ref, out_ref, h_ref):
    a = a_ref[0] + a_ref[1]
    mean = a / jnp.maximum(d_ref[...], 1.0)
    out = (jnp.dot(mean, wl_ref[...], preferred_element_type=f32)
           + bl_ref[...]
           + jnp.dot(x_ref[...], wr_ref[...], preferred_element_type=f32))
    out_ref[...] = out
    h_ref[...] = jnp.maximum(out, 0.0)


def _tc_conv1(a_ref, d_ref, h_ref, wl_ref, bl_ref, wr_ref, gs_ref, gc_ref,
              y_ref, g_ref):
    a = a_ref[0] + a_ref[1]
    mean = a / jnp.maximum(d_ref[...], 1.0)
    x2 = (jnp.dot(mean, wl_ref[...], preferred_element_type=f32)
          + bl_ref[...]
          + jnp.dot(h_ref[...], wr_ref[...], preferred_element_type=f32))
    m = jnp.max(x2, axis=1, keepdims=True)
    e = x2 - m
    lse = jnp.log(jnp.sum(jnp.exp(e), axis=1, keepdims=True))
    y_ref[...] = e - lse
    gs = gs_ref[0] + gs_ref[1]
    g_ref[...] = gs / jnp.maximum(gc_ref[...], 1.0)


ROWS_BLK = 1000
GRID = N // ROWS_BLK


def kernel(x, edge_index, cluster, Wl0, bl0, Wr0, Wl1, bl1, Wr1):
    src = edge_index[0]
    dst = edge_index[1]
    e_extra = E_PAD - E
    src_f = jnp.concatenate([src, jnp.zeros((e_extra,), i32)])
    dst_pad = N + (jnp.arange(e_extra, dtype=i32) % (N_PAD - N))
    dst_f = jnp.concatenate([dst, dst_pad])
    split = NS * EC0 * CH
    src0 = src_f[:split].reshape(NS, EC0, CH)
    dst0 = dst_f[:split].reshape(NS, EC0, CH)
    src1 = src_f[split:].reshape(NS, EC1, CH)
    dst1 = dst_f[split:].reshape(NS, EC1, CH)
    dst_p = dst_f.reshape(NW, EC, CH)
    p_extra = POOL_PAD - N
    clu_pad = N_CLUST + (jnp.arange(p_extra, dtype=i32) % (G_PAD - N_CLUST))
    clu_p = jnp.concatenate([cluster.astype(i32), clu_pad]).reshape(NW, PC, CH)
    zrow = jnp.zeros((CH, D), f32)

    deg_t, gcnt_t = _sc_counts(dst_p.reshape(NW, EPW), clu_p.reshape(NW, PPW))
    cat = jnp.concatenate([deg_t, gcnt_t], axis=1)
    ncat = N_PAD + G_PAD
    red = pl.pallas_call(
        _tc_reduce_counts,
        grid=(ncat // CH,),
        in_specs=[pl.BlockSpec((NW, CH), lambda i: (0, i))],
        out_specs=pl.BlockSpec((CH, 1), lambda i: (i, 0)),
        out_shape=jax.ShapeDtypeStruct((ncat, 1), f32),
    )(cat)
    deg_c = red[:N]
    gcnt_c = red[N_PAD:N_PAD + N_CLUST]

    agg0_p = _sc_agg(x, src0, dst0, src1, dst1, zrow)
    agg0_p = agg0_p[:, :N]

    wl0t = Wl0.T
    wr0t = Wr0.T
    bl0r = bl0.reshape(1, D)
    out, h = pl.pallas_call(
        _tc_conv0,
        grid=(GRID,),
        in_specs=[
            pl.BlockSpec((NC, ROWS_BLK, D), lambda i: (0, i, 0)),
            pl.BlockSpec((ROWS_BLK, 1), lambda i: (i, 0)),
            pl.BlockSpec((ROWS_BLK, D), lambda i: (i, 0)),
            pl.BlockSpec((D, D), lambda i: (0, 0)),
            pl.BlockSpec((1, D), lambda i: (0, 0)),
            pl.BlockSpec((D, D), lambda i: (0, 0)),
        ],
        out_specs=[
            pl.BlockSpec((ROWS_BLK, D), lambda i: (i, 0)),
            pl.BlockSpec((ROWS_BLK, D), lambda i: (i, 0)),
        ],
        out_shape=[
            jax.ShapeDtypeStruct((N, D), f32),
            jax.ShapeDtypeStruct((POOL_PAD, D), f32),
        ],
    )(agg0_p, deg_c, x, wl0t, bl0r, wr0t)

    agg1_p = _sc_agg(h, src0, dst0, src1, dst1, zrow)
    agg1_p = agg1_p[:, :N]
    g_p = _sc_pool(h, clu_p, zrow)
    g_p = g_p[:, :N_CLUST]

    wl1t = Wl1.T
    wr1t = Wr1.T
    bl1r = bl1.reshape(1, D)
    y, g = pl.pallas_call(
        _tc_conv1,
        grid=(GRID,),
        in_specs=[
            pl.BlockSpec((NC, ROWS_BLK, D), lambda i: (0, i, 0)),
            pl.BlockSpec((ROWS_BLK, 1), lambda i: (i, 0)),
            pl.BlockSpec((ROWS_BLK, D), lambda i: (i, 0)),
            pl.BlockSpec((D, D), lambda i: (0, 0)),
            pl.BlockSpec((1, D), lambda i: (0, 0)),
            pl.BlockSpec((D, D), lambda i: (0, 0)),
            pl.BlockSpec((NC, N_CLUST, D), lambda i: (0, 0, 0)),
            pl.BlockSpec((N_CLUST, 1), lambda i: (0, 0)),
        ],
        out_specs=[
            pl.BlockSpec((ROWS_BLK, D), lambda i: (i, 0)),
            pl.BlockSpec((N_CLUST, D), lambda i: (0, 0)),
        ],
        out_shape=[
            jax.ShapeDtypeStruct((N, D), f32),
            jax.ShapeDtypeStruct((N_CLUST, D), f32),
        ],
    )(agg1_p, deg_c, h, wl1t, bl1r, wr1t, g_p, gcnt_c)

    return (y, out, g)
